# Initial kernel scaffold; baseline (speedup 1.0000x reference)
import jax
import jax.numpy as jnp
from jax.experimental import pallas as pl


def kernel(x, lut):
    raise NotImplementedError("write your pallas kernel here")


# speedup vs baseline: 502.4594x; 502.4594x over previous
import functools

import jax
import jax.numpy as jnp
from jax import lax
from jax.experimental import pallas as pl
from jax.experimental.pallas import tpu as pltpu
from jax.experimental.pallas import tpu_sc as plsc

_LEVELS = 256
_SCALE = (_LEVELS - 1) / 2.0
_BIAS = _SCALE + 0.5

_N = 2 * 8192 * 2048
_NW = 32
_PER_W = _N // _NW
_CHUNK = 16384
_NCHUNK = _PER_W // _CHUNK


def _body(x_hbm, lut_hbm, out_hbm, lut_v, in_v, out_v):
    wid = lax.axis_index("s") * 2 + lax.axis_index("c")
    base = wid * _PER_W
    pltpu.sync_copy(lut_hbm, lut_v)

    @pl.loop(0, _NCHUNK)
    def _chunk(ci):
        off = pl.multiple_of(base + ci * _CHUNK, _CHUNK)
        pltpu.sync_copy(x_hbm.at[pl.ds(off, _CHUNK)], in_v)

        @pl.loop(0, _CHUNK // 16)
        def _vec(i):
            v = in_v[pl.ds(i * 16, 16)]
            v = jnp.minimum(jnp.maximum(v, -1.0), 1.0)
            idx = (v * _SCALE + _BIAS).astype(jnp.int32)
            out_v[pl.ds(i * 16, 16)] = plsc.load_gather(lut_v, [idx])

        pltpu.sync_copy(out_v, out_hbm.at[pl.ds(off, _CHUNK)])


@jax.jit
def _run(x_flat, lut):
    mesh = plsc.VectorSubcoreMesh(core_axis_name="c", subcore_axis_name="s")
    f = pl.kernel(
        _body,
        out_type=jax.ShapeDtypeStruct((_N,), jnp.float32),
        mesh=mesh,
        scratch_types=[
            pltpu.VMEM((_LEVELS,), jnp.float32),
            pltpu.VMEM((_CHUNK,), jnp.float32),
            pltpu.VMEM((_CHUNK,), jnp.float32),
        ],
        compiler_params=pltpu.CompilerParams(needs_layout_passes=False),
    )
    return f(x_flat, lut)


def kernel(x, lut):
    return _run(x.reshape(-1), lut).reshape(x.shape)


# speedup vs baseline: 935.4843x; 1.8618x over previous
import jax
import jax.numpy as jnp
from jax import lax
from jax.experimental import pallas as pl
from jax.experimental.pallas import tpu as pltpu
from jax.experimental.pallas import tpu_sc as plsc

_LEVELS = 256
_SCALE = (_LEVELS - 1) / 2.0
_BIAS = _SCALE + 0.5

_N = 2 * 8192 * 2048
_NW = 32
_PER_W = _N // _NW
_CHUNK = 16384
_NCHUNK = _PER_W // _CHUNK
_UNROLL = 8


def _body(x_hbm, lut_hbm, out_hbm, lut_v,
          in_v0, in_v1, out_v0, out_v1,
          isem0, isem1, osem0, osem1):
    wid = lax.axis_index("s") * 2 + lax.axis_index("c")
    base = wid * _PER_W
    pltpu.sync_copy(lut_hbm, lut_v)

    in_bufs = (in_v0, in_v1)
    out_bufs = (out_v0, out_v1)
    isems = (isem0, isem1)
    osems = (osem0, osem1)

    def off(ci):
        return pl.multiple_of(base + ci * _CHUNK, _CHUNK)

    def start_in(ci, b):
        pltpu.async_copy(x_hbm.at[pl.ds(off(ci), _CHUNK)], in_bufs[b], isems[b])

    def wait_in(b):
        pltpu.make_async_copy(x_hbm.at[pl.ds(0, _CHUNK)], in_bufs[b], isems[b]).wait()

    def start_out(ci, b):
        pltpu.async_copy(out_bufs[b], out_hbm.at[pl.ds(off(ci), _CHUNK)], osems[b])

    def wait_out(b):
        pltpu.make_async_copy(out_bufs[b], out_hbm.at[pl.ds(0, _CHUNK)], osems[b]).wait()

    def compute(b):
        inb, outb = in_bufs[b], out_bufs[b]

        @plsc.parallel_loop(0, _CHUNK, step=16, unroll=_UNROLL)
        def _vec(i):
            v = inb[pl.ds(i, 16)]
            v = jnp.minimum(jnp.maximum(v, -1.0), 1.0)
            idx = (v * _SCALE + _BIAS).astype(jnp.int32)
            outb[pl.ds(i, 16)] = plsc.load_gather(lut_v, [idx])

    start_in(0, 0)
    start_in(1, 1)
    for b in range(2):
        wait_in(b)
        compute(b)
        start_out(b, b)
        start_in(b + 2, b)

    @pl.loop(2, _NCHUNK - 2, step=2)
    def _main(ci):
        for b in range(2):
            cur = ci + b
            wait_in(b)
            wait_out(b)
            compute(b)
            start_out(cur, b)
            start_in(cur + 2, b)

    for b in range(2):
        wait_in(b)
        wait_out(b)
        compute(b)
        start_out(_NCHUNK - 2 + b, b)
    for b in range(2):
        wait_out(b)


@jax.jit
def _run(x_flat, lut):
    mesh = plsc.VectorSubcoreMesh(core_axis_name="c", subcore_axis_name="s")
    f = pl.kernel(
        _body,
        out_type=jax.ShapeDtypeStruct((_N,), jnp.float32),
        mesh=mesh,
        scratch_types=[
            pltpu.VMEM((_LEVELS,), jnp.float32),
            pltpu.VMEM((_CHUNK,), jnp.float32),
            pltpu.VMEM((_CHUNK,), jnp.float32),
            pltpu.VMEM((_CHUNK,), jnp.float32),
            pltpu.VMEM((_CHUNK,), jnp.float32),
            pltpu.SemaphoreType.DMA,
            pltpu.SemaphoreType.DMA,
            pltpu.SemaphoreType.DMA,
            pltpu.SemaphoreType.DMA,
        ],
        compiler_params=pltpu.CompilerParams(needs_layout_passes=False),
    )
    return f(x_flat, lut)


def kernel(x, lut):
    return _run(x.reshape(-1), lut).reshape(x.shape)


# speedup vs baseline: 2271.1209x; 2.4277x over previous
import jax
import jax.numpy as jnp
from jax import lax
from jax.experimental import pallas as pl
from jax.experimental.pallas import tpu as pltpu
from jax.experimental.pallas import tpu_sc as plsc

_LEVELS = 256
_SCALE = (_LEVELS - 1) / 2.0
_BIAS = _SCALE + 0.5

_B, _R, _C = 2, 8192, 2048
_NW = 32
_TPB = _NW // _B
_ROWS_PW = _R // _TPB
_CROWS = 8
_NCHUNK = _ROWS_PW // _CROWS
_UNROLL = 8


def _body(x_hbm, lut_hbm, out_hbm, lut_v,
          in_v0, in_v1, out_v0, out_v1,
          isem0, isem1, osem0, osem1):
    wid = lax.axis_index("s") * 2 + lax.axis_index("c")
    d0 = wid // _TPB
    row0 = (wid % _TPB) * _ROWS_PW
    pltpu.sync_copy(lut_hbm, lut_v)

    in_bufs = (in_v0, in_v1)
    out_bufs = (out_v0, out_v1)
    isems = (isem0, isem1)
    osems = (osem0, osem1)

    def row(ci):
        return pl.multiple_of(row0 + ci * _CROWS, _CROWS)

    def start_in(ci, b):
        pltpu.async_copy(x_hbm.at[d0, pl.ds(row(ci), _CROWS), :],
                         in_bufs[b], isems[b])

    def wait_in(b):
        pltpu.make_async_copy(x_hbm.at[0, pl.ds(0, _CROWS), :],
                              in_bufs[b], isems[b]).wait()

    def start_out(ci, b):
        pltpu.async_copy(out_bufs[b],
                         out_hbm.at[d0, pl.ds(row(ci), _CROWS), :], osems[b])

    def wait_out(b):
        pltpu.make_async_copy(out_bufs[b],
                              out_hbm.at[0, pl.ds(0, _CROWS), :], osems[b]).wait()

    def compute(b):
        inb, outb = in_bufs[b], out_bufs[b]
        for r in range(_CROWS):
            @plsc.parallel_loop(0, _C, step=16, unroll=_UNROLL)
            def _vec(i):
                v = inb[r, pl.ds(i, 16)]
                v = jnp.minimum(jnp.maximum(v, -1.0), 1.0)
                idx = (v * _SCALE + _BIAS).astype(jnp.int32)
                outb[r, pl.ds(i, 16)] = plsc.load_gather(lut_v, [idx])

    start_in(0, 0)
    start_in(1, 1)

    @pl.loop(0, _NCHUNK, step=2)
    def _main(ci):
        for b in range(2):
            cur = ci + b
            wait_in(b)

            @pl.when(cur >= 2)
            def _():
                wait_out(b)

            compute(b)
            start_out(cur, b)

            @pl.when(cur + 2 < _NCHUNK)
            def _():
                start_in(cur + 2, b)

    wait_out(0)
    wait_out(1)


@jax.jit
def kernel(x, lut):
    mesh = plsc.VectorSubcoreMesh(core_axis_name="c", subcore_axis_name="s")
    f = pl.kernel(
        _body,
        out_type=jax.ShapeDtypeStruct((_B, _R, _C), jnp.float32),
        mesh=mesh,
        scratch_types=[
            pltpu.VMEM((_LEVELS,), jnp.float32),
            pltpu.VMEM((_CROWS, _C), jnp.float32),
            pltpu.VMEM((_CROWS, _C), jnp.float32),
            pltpu.VMEM((_CROWS, _C), jnp.float32),
            pltpu.VMEM((_CROWS, _C), jnp.float32),
            pltpu.SemaphoreType.DMA,
            pltpu.SemaphoreType.DMA,
            pltpu.SemaphoreType.DMA,
            pltpu.SemaphoreType.DMA,
        ],
        compiler_params=pltpu.CompilerParams(needs_layout_passes=False),
    )
    return f(x, lut)
